# baseline (speedup 1.0000x reference)
import functools

import jax
import jax.numpy as jnp
from jax.experimental import pallas as pl
from jax.experimental.pallas import tpu as pltpu


def _vq_block_kernel(z_ref, e_ref, o_ref):
    z = z_ref[...]
    e = e_ref[...]
    zsq = jnp.sum(z * z, axis=1, keepdims=True)
    esq = jnp.sum(e * e, axis=1)
    scores = jax.lax.dot_general(
        z, e, (((1,), (1,)), ((), ())),
        preferred_element_type=jnp.float32)
    d = zsq + esq[None, :] - 2.0 * scores
    idx = jnp.argmin(d, axis=1)
    onehot = (jax.lax.broadcasted_iota(jnp.int32, d.shape, 1)
              == idx[:, None]).astype(jnp.float32)
    o_ref[...] = jax.lax.dot_general(
        onehot, e, (((1,), (0,)), ((), ())),
        precision=jax.lax.Precision.HIGHEST,
        preferred_element_type=jnp.float32)


@functools.partial(jax.jit, static_argnames=("interpret",))
def kernel(z, emb_weight, interpret=False):
    b, t, d = z.shape
    n = emb_weight.shape[0]
    zf = z.reshape(-1, d)
    m = zf.shape[0]
    blk = 2048
    out = pl.pallas_call(
        _vq_block_kernel,
        grid=(m // blk,),
        in_specs=[
            pl.BlockSpec((blk, d), lambda i: (i, 0)),
            pl.BlockSpec((n, d), lambda i: (0, 0)),
        ],
        out_specs=pl.BlockSpec((blk, d), lambda i: (i, 0)),
        out_shape=jax.ShapeDtypeStruct((m, d), jnp.float32),
        compiler_params=pltpu.CompilerParams(
            dimension_semantics=("parallel",)),
        interpret=interpret,
    )(zf, emb_weight)
    return out.reshape(z.shape)


# speedup vs baseline: 1.2415x; 1.2415x over previous
import functools

import jax
import jax.numpy as jnp
from jax.experimental import pallas as pl
from jax.experimental.pallas import tpu as pltpu


def _vq_block_kernel(z_ref, e_ref, o_ref):
    z = z_ref[...]
    e = e_ref[...]
    zsq = jnp.sum(z * z, axis=1, keepdims=True)
    esq = jnp.sum(e * e, axis=1)
    scores = jax.lax.dot_general(
        z, e, (((1,), (1,)), ((), ())),
        preferred_element_type=jnp.float32)
    d = zsq + esq[None, :] - 2.0 * scores
    idx = jnp.argmin(d, axis=1)
    onehot = (jax.lax.broadcasted_iota(jnp.int32, d.shape, 1)
              == idx[:, None]).astype(jnp.bfloat16)
    e_hi = e.astype(jnp.bfloat16)
    e_lo = (e - e_hi.astype(jnp.float32)).astype(jnp.bfloat16)
    g_hi = jax.lax.dot_general(
        onehot, e_hi, (((1,), (0,)), ((), ())),
        preferred_element_type=jnp.float32)
    g_lo = jax.lax.dot_general(
        onehot, e_lo, (((1,), (0,)), ((), ())),
        preferred_element_type=jnp.float32)
    o_ref[...] = g_hi + g_lo


@functools.partial(jax.jit, static_argnames=("interpret",))
def kernel(z, emb_weight, interpret=False):
    b, t, d = z.shape
    n = emb_weight.shape[0]
    zf = z.reshape(-1, d)
    m = zf.shape[0]
    blk = 2048
    out = pl.pallas_call(
        _vq_block_kernel,
        grid=(m // blk,),
        in_specs=[
            pl.BlockSpec((blk, d), lambda i: (i, 0)),
            pl.BlockSpec((n, d), lambda i: (0, 0)),
        ],
        out_specs=pl.BlockSpec((blk, d), lambda i: (i, 0)),
        out_shape=jax.ShapeDtypeStruct((m, d), jnp.float32),
        compiler_params=pltpu.CompilerParams(
            dimension_semantics=("parallel",)),
        interpret=interpret,
    )(zf, emb_weight)
    return out.reshape(z.shape)
